# baseline (speedup 1.0000x reference)
import functools

import jax
import jax.numpy as jnp
import numpy as np
from jax import lax
from jax.experimental import pallas as pl
from jax.experimental.pallas import tpu as pltpu, tpu_sc as plsc

B, T, C, H, W = 16, 16, 96, 114, 10
HW = H * W
BT = B * T
K_SP = max(1, int((1.0 - 0.333) * HW))
K_CH = max(1, int((1.0 - 0.333) * C))
NUM_APPLY = max(1, int(B * 0.333))

try:
    _perm = np.asarray(jax.random.permutation(jax.random.key(42), B))
except Exception:
    _perm = np.array([7, 4, 2, 5, 3, 6, 10, 11, 15, 8, 9, 13, 14, 0, 1, 12])
SEL = tuple(int(b) for b in _perm[:NUM_APPLY])
NONSEL = tuple(b for b in range(B) if b not in SEL)
NSEL = len(SEL)
P = NSEL * T
NROWS_NS = len(NONSEL) * T
TH = T // 2
NH = T // TH
RB = 4
RPW = -(-NROWS_NS // 32)

_F32_INF_BITS = 0x7F800000


def _selb(i):
    b = jnp.int32(SEL[0])
    for j in range(1, NSEL):
        b = jnp.where(i == j, jnp.int32(SEL[j]), b)
    return b


def _kth_bits(x_bits, k, axes, red_shape):
    lo = jnp.zeros(red_shape, jnp.int32)
    hi = jnp.full(red_shape, _F32_INF_BITS, jnp.int32)

    def body(_, carry):
        lo, hi = carry
        mid = lo + (hi - lo) // 2
        cnt = (x_bits <= mid).astype(jnp.int32)
        for ax in axes:
            cnt = jnp.sum(cnt, axis=ax, keepdims=True)
        ge = cnt >= k
        return jnp.where(ge, lo, mid + 1), jnp.where(ge, mid, hi)

    lo, hi = jax.lax.fori_loop(0, 31, body, (lo, hi))
    return hi


def _mask_body(g_ref, spm_ref, chm_ref):
    i = pl.program_id(0)
    h = pl.program_id(1)
    a = jnp.abs(g_ref[0])
    base = _selb(i) * T + h * TH
    spm_ref[pl.ds(base, TH)] = jnp.sum(a, axis=2)
    chm_ref[pl.ds(base, TH)] = jnp.sum(jnp.sum(a, axis=1), axis=2)

    @pl.when(jnp.logical_and(i == NSEL - 1, h == NH - 1))
    def _finalize():
        x = jnp.concatenate(
            [jax.lax.bitcast_convert_type(spm_ref[b * T:(b + 1) * T],
                                          jnp.int32) for b in SEL], axis=0)
        thr_sp = _kth_bits(x, K_SP, (2, 1), (P, 1, 1))
        spm = (x < thr_sp).astype(jnp.float32)

        y = jnp.concatenate(
            [jax.lax.bitcast_convert_type(chm_ref[b * T:(b + 1) * T],
                                          jnp.int32) for b in SEL], axis=0)
        thr_ch = _kth_bits(y, K_CH, (1,), (P, 1))
        chm = (y < thr_ch).astype(jnp.float32)

        for j, b in enumerate(SEL):
            spm_ref[b * T:(b + 1) * T] = spm[j * T:(j + 1) * T]
            chm_ref[b * T:(b + 1) * T] = chm[j * T:(j + 1) * T]


def _nonsel_row(i):
    idx = i // T
    b = jnp.int32(NONSEL[0])
    for j in range(1, len(NONSEL)):
        b = jnp.where(idx == j, jnp.int32(NONSEL[j]), b)
    return b * T + i % T


_sc_mesh = plsc.VectorSubcoreMesh(core_axis_name="c", subcore_axis_name="s")


@functools.partial(
    pl.kernel, mesh=_sc_mesh,
    out_type=jax.ShapeDtypeStruct((BT, W, C, H), jnp.float32),
)
def _sc_copy(z_hbm, out_hbm):
    wid = lax.axis_index("s") * 2 + lax.axis_index("c")
    for j in range(RPW):
        i = wid * RPW + j

        @pl.when(i < NROWS_NS)
        def _():
            g = _nonsel_row(i)
            pltpu.sync_copy(z_hbm.at[pl.ds(g, 1)], out_hbm.at[pl.ds(g, 1)])


def _apply_im(q):
    blk = jnp.int32(SEL[0] * (T // RB))
    for j in range(1, NSEL):
        blk = jnp.where(q // (T // RB) == j, jnp.int32(SEL[j] * (T // RB)), blk)
    return blk + q % (T // RB)


def _apply_body(z_ref, spm_ref, chm_ref, o1_ref, out_ref):
    del o1_ref
    spv = spm_ref[...]
    for r in range(RB):
        ch_col = jax.lax.dot_general(
            chm_ref[r], jnp.ones((1, 1), jnp.float32),
            (((0,), (0,)), ((), ())),
            preferred_element_type=jnp.float32)
        for w in range(W):
            out_ref[r, w] = z_ref[r, w] * (ch_col * spv[r, w:w + 1, :])


def kernel(z_local, gradient):
    f32 = jnp.float32
    gt = gradient.transpose(0, 1, 4, 2, 3)
    zt = z_local.transpose(0, 1, 4, 2, 3).reshape(BT, W, C, H)

    out1 = _sc_copy(zt)

    spm, chm = pl.pallas_call(
        _mask_body,
        grid=(NSEL, NH),
        in_specs=[pl.BlockSpec((1, TH, W, C, H),
                               lambda i, h: (_selb(i), h, 0, 0, 0))],
        out_specs=[
            pl.BlockSpec((BT, W, H), lambda i, h: (0, 0, 0)),
            pl.BlockSpec((BT, C), lambda i, h: (0, 0)),
        ],
        out_shape=[
            jax.ShapeDtypeStruct((BT, W, H), f32),
            jax.ShapeDtypeStruct((BT, C), f32),
        ],
    )(gt)

    out = pl.pallas_call(
        _apply_body,
        grid=(P // RB,),
        in_specs=[
            pl.BlockSpec((RB, W, C, H), lambda q: (_apply_im(q), 0, 0, 0)),
            pl.BlockSpec((RB, W, H), lambda q: (_apply_im(q), 0, 0)),
            pl.BlockSpec((RB, 1, C), lambda q: (_apply_im(q), 0, 0)),
            pl.BlockSpec(memory_space=pl.ANY),
        ],
        out_specs=pl.BlockSpec((RB, W, C, H), lambda q: (_apply_im(q), 0, 0, 0)),
        out_shape=jax.ShapeDtypeStruct((BT, W, C, H), f32),
        input_output_aliases={3: 0},
    )(zt, spm, chm.reshape(BT, 1, C), out1)

    return out.reshape(B, T, W, C, H).transpose(0, 1, 3, 4, 2)


# speedup vs baseline: 21.8977x; 21.8977x over previous
import jax
import jax.numpy as jnp
import numpy as np
from jax.experimental import pallas as pl
from jax.experimental.pallas import tpu as pltpu

B, T, C, H, W = 16, 16, 96, 114, 10
HW = H * W
BT = B * T
K_SP = max(1, int((1.0 - 0.333) * HW))
K_CH = max(1, int((1.0 - 0.333) * C))
NUM_APPLY = max(1, int(B * 0.333))

try:
    _perm = np.asarray(jax.random.permutation(jax.random.key(42), B))
except Exception:
    _perm = np.array([7, 4, 2, 5, 3, 6, 10, 11, 15, 8, 9, 13, 14, 0, 1, 12])
SEL = tuple(int(b) for b in _perm[:NUM_APPLY])
NONSEL = tuple(b for b in range(B) if b not in SEL)
NSEL = len(SEL)
P = NSEL * T
TH = T // 2
NH = T // TH
NPRE = NSEL * NH
RB = 8

_F32_INF_BITS = 0x7F800000


def _selb(i):
    b = jnp.int32(SEL[0])
    for j in range(1, NSEL):
        b = jnp.where(i == j, jnp.int32(SEL[j]), b)
    return b


def _kth_bits(x_bits, k, axes, red_shape):
    lo = jnp.zeros(red_shape, jnp.int32)
    hi = jnp.full(red_shape, _F32_INF_BITS, jnp.int32)

    def body(_, carry):
        lo, hi = carry
        mid = lo + (hi - lo) // 2
        cnt = (x_bits <= mid).astype(jnp.int32)
        for ax in axes:
            cnt = jnp.sum(cnt, axis=ax, keepdims=True)
        ge = cnt >= k
        return jnp.where(ge, lo, mid + 1), jnp.where(ge, mid, hi)

    lo, hi = jax.lax.fori_loop(0, 31, body, (lo, hi))
    return hi


def _fused_body(g_ref, z_ref, out_ref, spm_s, chm_s):
    p = pl.program_id(0)

    @pl.when(p < NPRE)
    def _reduce():
        i = p // NH
        h = p % NH
        a = jnp.abs(g_ref[0])
        base = _selb(i) * T + h * TH
        spm_s[pl.ds(base, TH)] = jnp.sum(a, axis=2)
        chm_s[pl.ds(base, TH)] = jnp.sum(jnp.sum(a, axis=1), axis=2)

    @pl.when(p == NPRE - 1)
    def _finalize():
        x = jnp.concatenate(
            [jax.lax.bitcast_convert_type(spm_s[b * T:(b + 1) * T],
                                          jnp.int32) for b in SEL], axis=0)
        thr_sp = _kth_bits(x, K_SP, (2, 1), (P, 1, 1))
        spm = (x < thr_sp).astype(jnp.float32)

        y = jnp.concatenate(
            [jax.lax.bitcast_convert_type(chm_s[b * T:(b + 1) * T],
                                          jnp.int32) for b in SEL], axis=0)
        thr_ch = _kth_bits(y, K_CH, (1,), (P, 1))
        chm = (y < thr_ch).astype(jnp.float32)

        for j, b in enumerate(SEL):
            spm_s[b * T:(b + 1) * T] = spm[j * T:(j + 1) * T]
            chm_s[b * T:(b + 1) * T] = chm[j * T:(j + 1) * T]
        for b in NONSEL:
            spm_s[b * T:(b + 1) * T] = jnp.ones((T, W, H), jnp.float32)
            chm_s[b * T:(b + 1) * T] = jnp.ones((T, C), jnp.float32)

    @pl.when(p >= NPRE)
    def _apply():
        base = (p - NPRE) * RB
        spv = spm_s[pl.ds(base, RB)]
        chv = chm_s[pl.ds(base, RB)]
        for r in range(RB):
            ch_col = jax.lax.dot_general(
                chv[r:r + 1], jnp.ones((1, 1), jnp.float32),
                (((0,), (0,)), ((), ())),
                preferred_element_type=jnp.float32)
            for w in range(W):
                out_ref[r, w] = z_ref[r, w] * (ch_col * spv[r, w:w + 1, :])


def _g_im(p):
    pp = jnp.minimum(p, NPRE - 1)
    return (_selb(pp // NH), pp % NH, 0, 0, 0)


def _z_im(p):
    return (jnp.maximum(p - NPRE, 0), 0, 0, 0)


def kernel(z_local, gradient):
    f32 = jnp.float32
    gt = gradient.transpose(0, 1, 4, 2, 3)
    zt = z_local.transpose(0, 1, 4, 2, 3).reshape(BT, W, C, H)

    out = pl.pallas_call(
        _fused_body,
        grid=(NPRE + BT // RB,),
        in_specs=[
            pl.BlockSpec((1, TH, W, C, H), _g_im),
            pl.BlockSpec((RB, W, C, H), _z_im),
        ],
        out_specs=pl.BlockSpec((RB, W, C, H), _z_im),
        out_shape=jax.ShapeDtypeStruct((BT, W, C, H), f32),
        scratch_shapes=[
            pltpu.VMEM((BT, W, H), f32),
            pltpu.VMEM((BT, C), f32),
        ],
    )(gt, zt)

    return out.reshape(B, T, W, C, H).transpose(0, 1, 3, 4, 2)


# speedup vs baseline: 22.3436x; 1.0204x over previous
import jax
import jax.numpy as jnp
import numpy as np
from jax.experimental import pallas as pl
from jax.experimental.pallas import tpu as pltpu

B, T, C, H, W = 16, 16, 96, 114, 10
HW = H * W
BT = B * T
K_SP = max(1, int((1.0 - 0.333) * HW))
K_CH = max(1, int((1.0 - 0.333) * C))
NUM_APPLY = max(1, int(B * 0.333))

try:
    _perm = np.asarray(jax.random.permutation(jax.random.key(42), B))
except Exception:
    _perm = np.array([7, 4, 2, 5, 3, 6, 10, 11, 15, 8, 9, 13, 14, 0, 1, 12])
SEL = tuple(int(b) for b in _perm[:NUM_APPLY])
NONSEL = tuple(b for b in range(B) if b not in SEL)
NSEL = len(SEL)
P = NSEL * T
TH = T // 2
NH = T // TH
NPRE = NSEL * NH
RB = 16

_F32_INF_BITS = 0x7F800000


def _selb(i):
    b = jnp.int32(SEL[0])
    for j in range(1, NSEL):
        b = jnp.where(i == j, jnp.int32(SEL[j]), b)
    return b


def _kth_bits(x_bits, k, axes, red_shape):
    lo = jnp.zeros(red_shape, jnp.int32)
    hi = jnp.full(red_shape, _F32_INF_BITS, jnp.int32)

    def body(_, carry):
        lo, hi = carry
        mid = lo + (hi - lo) // 2
        cnt = (x_bits <= mid).astype(jnp.int32)
        for ax in axes:
            cnt = jnp.sum(cnt, axis=ax, keepdims=True)
        ge = cnt >= k
        return jnp.where(ge, lo, mid + 1), jnp.where(ge, mid, hi)

    lo, hi = jax.lax.fori_loop(0, 31, body, (lo, hi))
    return hi


def _fused_body(g_ref, z_ref, out_ref, spm_s, chm_s):
    p = pl.program_id(0)

    @pl.when(p < NPRE)
    def _reduce():
        i = p // NH
        h = p % NH
        a = jnp.abs(g_ref[0])
        base = _selb(i) * T + h * TH
        spm_s[pl.ds(base, TH)] = jnp.sum(a, axis=2)
        chm_s[pl.ds(base, TH)] = jnp.sum(jnp.sum(a, axis=1), axis=2)

    @pl.when(p == NPRE - 1)
    def _finalize():
        x = jnp.concatenate(
            [jax.lax.bitcast_convert_type(spm_s[b * T:(b + 1) * T],
                                          jnp.int32) for b in SEL], axis=0)
        thr_sp = _kth_bits(x, K_SP, (2, 1), (P, 1, 1))
        spm = (x < thr_sp).astype(jnp.float32)

        y = jnp.concatenate(
            [jax.lax.bitcast_convert_type(chm_s[b * T:(b + 1) * T],
                                          jnp.int32) for b in SEL], axis=0)
        thr_ch = _kth_bits(y, K_CH, (1,), (P, 1))
        chm = (y < thr_ch).astype(jnp.float32)

        for j, b in enumerate(SEL):
            spm_s[b * T:(b + 1) * T] = spm[j * T:(j + 1) * T]
            chm_s[b * T:(b + 1) * T] = chm[j * T:(j + 1) * T]
        for b in NONSEL:
            spm_s[b * T:(b + 1) * T] = jnp.ones((T, W, H), jnp.float32)
            chm_s[b * T:(b + 1) * T] = jnp.ones((T, C), jnp.float32)

    @pl.when(p >= NPRE)
    def _apply():
        base = (p - NPRE) * RB
        spv = spm_s[pl.ds(base, RB)]
        chv = chm_s[pl.ds(base, RB)]
        for r in range(RB):
            ch_col = jax.lax.dot_general(
                chv[r:r + 1], jnp.ones((1, 1), jnp.float32),
                (((0,), (0,)), ((), ())),
                preferred_element_type=jnp.float32)
            for w in range(W):
                out_ref[r, w] = z_ref[r, w] * (ch_col * spv[r, w:w + 1, :])


def _g_im(p):
    pp = jnp.minimum(p, NPRE - 1)
    return (_selb(pp // NH), pp % NH, 0, 0, 0)


def _z_im(p):
    return (jnp.maximum(p - NPRE, 0), 0, 0, 0)


def kernel(z_local, gradient):
    f32 = jnp.float32
    gt = gradient.transpose(0, 1, 4, 2, 3)
    zt = z_local.transpose(0, 1, 4, 2, 3).reshape(BT, W, C, H)

    out = pl.pallas_call(
        _fused_body,
        grid=(NPRE + BT // RB,),
        in_specs=[
            pl.BlockSpec((1, TH, W, C, H), _g_im),
            pl.BlockSpec((RB, W, C, H), _z_im),
        ],
        out_specs=pl.BlockSpec((RB, W, C, H), _z_im),
        out_shape=jax.ShapeDtypeStruct((BT, W, C, H), f32),
        scratch_shapes=[
            pltpu.VMEM((BT, W, H), f32),
            pltpu.VMEM((BT, C), f32),
        ],
    )(gt, zt)

    return out.reshape(B, T, W, C, H).transpose(0, 1, 3, 4, 2)
